# Initial kernel scaffold; baseline (speedup 1.0000x reference)
import jax
import jax.numpy as jnp
from jax.experimental import pallas as pl


def kernel(x, wq, bq, wk, bk, wv, bv):
    raise NotImplementedError("write your pallas kernel here")


# speedup vs baseline: 4.2453x; 4.2453x over previous
import functools
import math

import jax
import jax.numpy as jnp
from jax.experimental import pallas as pl
from jax.experimental.pallas import tpu as pltpu

_LOG2E = 1.4426950408889634
_NEG_BIG = -1e30

_TQ = 1024
_TK = 2048


def _round_up(a, b):
    return ((a + b - 1) // b) * b


def _absmax_kernel(x_ref, o_ref):
    o_ref[0, 0] = jnp.max(jnp.abs(x_ref[...]))


def _prep_kernel(x_ref, w_ref, b_ref, xinf_ref, qp_ref, k4_ref, v2_ref):
    x = x_ref[...]
    g = jnp.dot(x, w_ref[...], preferred_element_type=jnp.float32) + b_ref[...]
    u = g[:, 0:3]
    t = g[:, 3:4]
    v = g[:, 4:5]
    m = jnp.sum(jnp.abs(u), axis=1, keepdims=True) * xinf_ref[0, 0]
    qp_ref[...] = jnp.concatenate([u, t - m], axis=1) * _LOG2E
    ones = jnp.ones_like(t)
    k4_ref[...] = jnp.concatenate([x, ones], axis=1)
    v2_ref[...] = jnp.concatenate([v, ones], axis=1)


def _attn_kernel(n_valid, masked, qp_ref, k4_ref, v2_ref, z_ref, acc_ref):
    ki = pl.program_id(1)

    @pl.when(ki == 0)
    def _():
        acc_ref[...] = jnp.zeros_like(acc_ref)

    s = jax.lax.dot_general(qp_ref[...], k4_ref[...], (((1,), (1,)), ((), ())),
                            preferred_element_type=jnp.float32)
    if masked:
        col = ki * s.shape[1] + jax.lax.broadcasted_iota(jnp.int32, s.shape, 1)
        s = jnp.where(col < n_valid, s, _NEG_BIG)
    p = jnp.exp2(s)
    acc_ref[...] += jnp.dot(p, v2_ref[...],
                            preferred_element_type=jnp.float32)

    @pl.when(ki == pl.num_programs(1) - 1)
    def _():
        a = acc_ref[...]
        z_ref[...] = a[:, 0:1] * pl.reciprocal(a[:, 1:2], approx=True)


def _final_softmax_kernel(n_valid, masked, z_ref, o_ref):
    z = z_ref[...]
    if masked:
        col = jax.lax.broadcasted_iota(jnp.int32, z.shape, 1)
        z = jnp.where(col < n_valid, z, _NEG_BIG)
    e = jnp.exp(z - jnp.max(z, axis=-1, keepdims=True))
    o_ref[...] = e * pl.reciprocal(jnp.sum(e, axis=-1, keepdims=True),
                                   approx=False)


def kernel(x, wq, bq, wk, bk, wv, bv):
    n = x.shape[0]
    tq = min(_TQ, _round_up(max(n, 1), 8))
    tk = min(_TK, _round_up(max(n, 1), 8))
    n_pad = _round_up(n, (tq * tk) // math.gcd(tq, tk))
    masked = n_pad != n

    x32 = x.astype(jnp.float32)
    if masked:
        x32 = jnp.pad(x32, ((0, n_pad - n), (0, 0)))

    scale = 1.0 / math.sqrt(8.0)
    wqT = wq.T.astype(jnp.float32) * scale
    bqs = bq.astype(jnp.float32) * scale
    wk32 = wk.astype(jnp.float32)
    bk32 = bk.astype(jnp.float32)
    a_mat = wqT @ wk32
    c_vec = bqs @ wk32
    d_vec = wqT @ bk32
    e_sc = bqs @ bk32
    w5 = jnp.concatenate(
        [a_mat, d_vec[:, None], wv.T.astype(jnp.float32)], axis=1)
    b5 = jnp.concatenate(
        [c_vec, e_sc[None], bv.astype(jnp.float32)]).reshape(1, 5)

    flen = _round_up(n_pad * 3, 1024)
    fv = x32.reshape(-1)
    if flen != n_pad * 3:
        fv = jnp.pad(fv, (0, flen - n_pad * 3))
    flat = fv.reshape(flen // 128, 128)
    xinf = pl.pallas_call(
        _absmax_kernel,
        out_shape=jax.ShapeDtypeStruct((1, 1), jnp.float32),
        in_specs=[pl.BlockSpec(memory_space=pltpu.MemorySpace.VMEM)],
        out_specs=pl.BlockSpec(memory_space=pltpu.MemorySpace.SMEM),
        compiler_params=pltpu.CompilerParams(
            vmem_limit_bytes=48 * 1024 * 1024),
    )(flat)

    tp = min(4096, n_pad)
    qp, k4, v2 = pl.pallas_call(
        _prep_kernel,
        out_shape=(
            jax.ShapeDtypeStruct((n_pad, 4), jnp.float32),
            jax.ShapeDtypeStruct((n_pad, 4), jnp.float32),
            jax.ShapeDtypeStruct((n_pad, 2), jnp.float32),
        ),
        grid=(n_pad // tp,),
        in_specs=[
            pl.BlockSpec((tp, 3), lambda i: (i, 0)),
            pl.BlockSpec((3, 5), lambda i: (0, 0)),
            pl.BlockSpec((1, 5), lambda i: (0, 0)),
            pl.BlockSpec(memory_space=pltpu.MemorySpace.SMEM),
        ],
        out_specs=(
            pl.BlockSpec((tp, 4), lambda i: (i, 0)),
            pl.BlockSpec((tp, 4), lambda i: (i, 0)),
            pl.BlockSpec((tp, 2), lambda i: (i, 0)),
        ),
        compiler_params=pltpu.CompilerParams(
            dimension_semantics=("arbitrary",)),
    )(x32, w5, b5, xinf)

    n_q = n_pad // tq
    n_kv = n_pad // tk
    cost = pl.CostEstimate(
        flops=2 * n_pad * n_pad * (4 + 2),
        transcendentals=n_pad * n_pad,
        bytes_accessed=4 * (n_pad * 4 + n_q * n_pad * 6 + n_pad),
    )
    z = pl.pallas_call(
        functools.partial(_attn_kernel, n, masked),
        out_shape=jax.ShapeDtypeStruct((n_pad, 1), jnp.float32),
        grid=(n_q, n_kv),
        in_specs=[
            pl.BlockSpec((tq, 4), lambda qi, ki: (qi, 0)),
            pl.BlockSpec((tk, 4), lambda qi, ki: (ki, 0)),
            pl.BlockSpec((tk, 2), lambda qi, ki: (ki, 0)),
        ],
        out_specs=pl.BlockSpec((tq, 1), lambda qi, ki: (qi, 0)),
        scratch_shapes=[pltpu.VMEM((tq, 2), jnp.float32)],
        compiler_params=pltpu.CompilerParams(
            dimension_semantics=("parallel", "arbitrary"),
            vmem_limit_bytes=48 * 1024 * 1024,
        ),
        cost_estimate=cost,
    )(qp, k4, v2)

    z_lane = z.reshape(1, n_pad)
    out = pl.pallas_call(
        functools.partial(_final_softmax_kernel, n, masked),
        out_shape=jax.ShapeDtypeStruct((1, n_pad), jnp.float32),
        in_specs=[pl.BlockSpec(memory_space=pltpu.MemorySpace.VMEM)],
        out_specs=pl.BlockSpec(memory_space=pltpu.MemorySpace.VMEM),
    )(z_lane)

    return out[0, :n]
